# Initial kernel scaffold; baseline (speedup 1.0000x reference)
import jax
import jax.numpy as jnp
from jax.experimental import pallas as pl


def kernel(x, edge_index, batch, edge_attr, conv1_w, conv1_b, conv2_w, conv2_b, conv3_w, conv3_b, fc_w, fc_b):
    raise NotImplementedError("write your pallas kernel here")


# speedup vs baseline: 18.2751x; 18.2751x over previous
import functools

import jax
import jax.numpy as jnp
from jax import lax
from jax.experimental import pallas as pl
from jax.experimental.pallas import tpu as pltpu
from jax.experimental.pallas import tpu_sc as plsc

N = 10000
E = 320000
D = 128
H = 8
G = 64

NC = 2
NS = 16
NW = NC * NS
EW = E // NW
CH = 2000
NCHUNK = EW // CH
RPA = 624
TOFF = NS * RPA
TAIL = N - TOFF

NB = 1000
NBLK = N // NB

_mesh = plsc.VectorSubcoreMesh(core_axis_name="c", subcore_axis_name="s")
_sc_params = pltpu.CompilerParams(needs_layout_passes=False, use_tc_tiling_on_sc=False)


def _f32(shape):
    return jax.ShapeDtypeStruct(shape, jnp.float32)


@functools.partial(
    pl.kernel,
    out_type=[_f32((NC * N,)), _f32((E * H,))],
    mesh=_mesh,
    scratch_types=[
        pltpu.VMEM((CH,), jnp.int32),
        pltpu.VMEM((CH,), jnp.float32),
        pltpu.VMEM((CH * H,), jnp.float32),
        pltpu.VMEM((RPA,), jnp.float32),
        pltpu.VMEM_SHARED((N,), jnp.float32),
        pltpu.SemaphoreType.DMA,
    ],
    compiler_params=_sc_params,
)
def _deg_ea8(col_h, ea_h, zn_h, d_h, ea8_h, colbuf, eabuf, ea8buf, dbuf, deg_sp, sem):
    cid = lax.axis_index("c")
    sid = lax.axis_index("s")
    wid = cid * NS + sid

    @pl.when(sid == 0)
    def _():
        pltpu.sync_copy(zn_h, deg_sp)

    plsc.subcore_barrier()
    lanes = lax.iota(jnp.int32, 16)
    hi = lanes // H

    def chunk(k, carry):
        eoff = wid * EW + k * CH
        pltpu.sync_copy(col_h.at[pl.ds(eoff, CH)], colbuf)
        pltpu.sync_copy(ea_h.at[pl.ds(eoff, CH)], eabuf)

        def expand(i, c2):
            e2 = plsc.load_gather(eabuf, [hi + 2 * i])
            plsc.store_scatter(ea8buf, [lanes + 16 * i], e2)
            return c2

        lax.fori_loop(0, CH // 2, expand, 0, unroll=4)
        pltpu.sync_copy(ea8buf, ea8_h.at[pl.ds(eoff * H, CH * H)])
        pltpu.sync_copy(eabuf, deg_sp.at[colbuf], add=True)
        return carry

    lax.fori_loop(0, NCHUNK, chunk, 0)
    plsc.subcore_barrier()
    pltpu.sync_copy(deg_sp.at[pl.ds(sid * RPA, RPA)], dbuf)
    pltpu.sync_copy(dbuf, d_h.at[pl.ds(cid * N + sid * RPA, RPA)])

    @pl.when(sid == NS - 1)
    def _():
        pltpu.sync_copy(deg_sp.at[pl.ds(TOFF, TAIL)], dbuf.at[pl.ds(0, TAIL)])
        pltpu.sync_copy(dbuf.at[pl.ds(0, TAIL)],
                        d_h.at[pl.ds(cid * N + TOFF, TAIL)])


def _make_pass(two_inputs):
    scratch = [
        pltpu.VMEM((CH,), jnp.int32),
        pltpu.VMEM((CH,), jnp.int32),
        pltpu.VMEM((CH * H,), jnp.float32),
        pltpu.VMEM((CH, H), jnp.float32),
        pltpu.VMEM((RPA, H), jnp.float32),
        pltpu.VMEM((RPA, H), jnp.float32),
        pltpu.VMEM((RPA, H), jnp.float32),
        pltpu.VMEM_SHARED((N, H), jnp.float32),
        pltpu.VMEM_SHARED((N, H), jnp.float32),
        pltpu.SemaphoreType.DMA,
    ]

    def body(*refs):
        if two_inputs:
            (p_h, dg8_h, row_h, col_h, ea8_h, zn8_h, q_h,
             rowbuf, colbuf, eabuf, gbuf, pb0, pb1, dbuf, in_sp, out_sp, sem) = refs
        else:
            (p_h, row_h, col_h, ea8_h, zn8_h, q_h,
             rowbuf, colbuf, eabuf, gbuf, pb0, pb1, dbuf, in_sp, out_sp, sem) = refs
        cid = lax.axis_index("c")
        sid = lax.axis_index("s")
        wid = cid * NS + sid
        lanes = lax.iota(jnp.int32, 16)
        ci_base = lanes // H
        jmod = lanes % H

        @pl.when(sid == 0)
        def _():
            pltpu.sync_copy(zn8_h, out_sp)
            if not two_inputs:
                pltpu.sync_copy(p_h, in_sp)

        def prologue(base, rows):
            pltpu.sync_copy(p_h.at[0, pl.ds(base, rows), :],
                            pb0.at[pl.ds(0, rows), :])
            pltpu.sync_copy(p_h.at[1, pl.ds(base, rows), :],
                            pb1.at[pl.ds(0, rows), :])
            pltpu.sync_copy(dg8_h.at[pl.ds(base, rows), :],
                            dbuf.at[pl.ds(0, rows), :])

            def comb(i, c):
                ci = ci_base + 2 * i
                v = (plsc.load_gather(pb0, [ci, jmod])
                     + plsc.load_gather(pb1, [ci, jmod]))
                v = v * plsc.load_gather(dbuf, [ci, jmod])
                plsc.store_scatter(pb0, [ci, jmod], v)
                return c

            lax.fori_loop(0, rows // 2, comb, 0, unroll=4)
            pltpu.sync_copy(pb0.at[pl.ds(0, rows), :],
                            in_sp.at[pl.ds(base, rows), :])

        if two_inputs:
            prologue(sid * RPA, RPA)

            @pl.when(sid == NS - 1)
            def _():
                prologue(TOFF, TAIL)

        plsc.subcore_barrier()

        def chunk(k, c):
            eoff = wid * EW + k * CH
            pltpu.sync_copy(row_h.at[pl.ds(eoff, CH)], rowbuf)
            pltpu.sync_copy(col_h.at[pl.ds(eoff, CH)], colbuf)
            pltpu.sync_copy(ea8_h.at[pl.ds(eoff * H, CH * H)], eabuf)
            pltpu.async_copy(in_sp.at[rowbuf], gbuf, sem).wait()

            def scale(i, c2):
                ci = ci_base + 2 * i
                g = plsc.load_gather(gbuf, [ci, jmod])
                e = plsc.load_gather(eabuf, [lanes + 16 * i])
                plsc.store_scatter(gbuf, [ci, jmod], g * e)
                return c2

            lax.fori_loop(0, CH // 2, scale, 0, unroll=4)
            pltpu.sync_copy(gbuf, out_sp.at[colbuf], add=True)
            return c

        lax.fori_loop(0, NCHUNK, chunk, 0)
        plsc.subcore_barrier()
        pltpu.sync_copy(out_sp.at[pl.ds(sid * RPA, RPA), :], pb0)
        pltpu.sync_copy(pb0, q_h.at[cid, pl.ds(sid * RPA, RPA), :])

        @pl.when(sid == NS - 1)
        def _():
            pltpu.sync_copy(out_sp.at[pl.ds(TOFF, TAIL), :],
                            pb1.at[pl.ds(0, TAIL), :])
            pltpu.sync_copy(pb1.at[pl.ds(0, TAIL), :],
                            q_h.at[cid, pl.ds(TOFF, TAIL), :])

    return pl.kernel(body, out_type=_f32((NC, N, H)), mesh=_mesh,
                     scratch_types=scratch, compiler_params=_sc_params)


_pass_first = _make_pass(False)
_pass_mid = _make_pass(True)


def _proj_body(x_ref, d0_ref, d1_ref, w_ref, y0_ref, z1_ref, z2_ref, z3_ref,
               di8_ref, dg8_ref):
    deg = d0_ref[...] + d1_ref[...]
    dinv = jnp.where(deg > 0, lax.rsqrt(jnp.maximum(deg, 1e-12)), 0.0)
    dg = dinv * dinv
    yz = lax.dot_general(x_ref[...], w_ref[...], (((1,), (1,)), ((), ())),
                         preferred_element_type=jnp.float32)
    y0_ref[...] = yz[:, 0:H]
    z1_ref[...] = dinv * yz[:, H:2 * H]
    z2_ref[...] = dinv * yz[:, 2 * H:3 * H]
    z3_ref[...] = dinv * yz[:, 3 * H:4 * H]
    di8_ref[...] = jnp.broadcast_to(dinv, (NB, H))
    dg8_ref[...] = jnp.broadcast_to(dg, (NB, H))


_proj = pl.pallas_call(
    _proj_body,
    grid=(NBLK,),
    in_specs=[
        pl.BlockSpec((NB, D), lambda i: (i, 0)),
        pl.BlockSpec((NB, 1), lambda i: (i, 0)),
        pl.BlockSpec((NB, 1), lambda i: (i, 0)),
        pl.BlockSpec((4 * H, D), lambda i: (0, 0)),
    ],
    out_specs=[pl.BlockSpec((NB, H), lambda i: (i, 0))] * 6,
    out_shape=[_f32((N, H))] * 6,
)


def _l1_body(y0, u1a, u1b, u2a, u2b, u3a, u3b, di8, b, h_ref, g_ref):
    s = (u1a[...] + u1b[...]) + (u2a[...] + u2b[...]) + (u3a[...] + u3b[...])
    h = jnp.maximum(y0[...] + di8[...] * s + b[...], 0.0)
    h_ref[...] = h
    g_ref[...] = di8[...] * h


_l1 = pl.pallas_call(
    _l1_body,
    grid=(NBLK,),
    in_specs=[pl.BlockSpec((NB, H), lambda i: (i, 0))] * 8
    + [pl.BlockSpec((1, H), lambda i: (0, 0))],
    out_specs=[pl.BlockSpec((NB, H), lambda i: (i, 0))] * 2,
    out_shape=[_f32((N, H))] * 2,
)


def _dotT(a, w):
    return lax.dot_general(a, w, (((1,), (1,)), ((), ())),
                           preferred_element_type=jnp.float32)


def _lh_body(hp, v1a, v1b, v2a, v2b, v3a, v3b, di8, w_ref, b, h_ref, g_ref):
    w = w_ref[...]
    t = (_dotT(v1a[...] + v1b[...], w[H:2 * H])
         + _dotT(v2a[...] + v2b[...], w[2 * H:3 * H])
         + _dotT(v3a[...] + v3b[...], w[3 * H:4 * H]))
    h = jnp.maximum(_dotT(hp[...], w[0:H]) + di8[...] * t + b[...], 0.0)
    h_ref[...] = h
    g_ref[...] = di8[...] * h


_lh = pl.pallas_call(
    _lh_body,
    grid=(NBLK,),
    in_specs=[pl.BlockSpec((NB, H), lambda i: (i, 0))] * 8
    + [pl.BlockSpec((4 * H, H), lambda i: (0, 0)),
       pl.BlockSpec((1, H), lambda i: (0, 0))],
    out_specs=[pl.BlockSpec((NB, H), lambda i: (i, 0))] * 2,
    out_shape=[_f32((N, H))] * 2,
)


def _fin_body(h1r, h2r, v1a, v1b, v2a, v2b, v3a, v3b, di8, w_ref, b, batr,
              fcw, fcb, out_ref, smax, ssum, scnt):
    i = pl.program_id(0)

    @pl.when(i == 0)
    def _():
        smax[...] = jnp.full((G, 128), -jnp.inf, jnp.float32)
        ssum[...] = jnp.zeros((G, 128), jnp.float32)
        scnt[...] = jnp.zeros((G, 128), jnp.float32)

    w = w_ref[...]
    t = (_dotT(v1a[...] + v1b[...], w[H:2 * H])
         + _dotT(v2a[...] + v2b[...], w[2 * H:3 * H])
         + _dotT(v3a[...] + v3b[...], w[3 * H:4 * H]))
    h3 = jnp.maximum(_dotT(h2r[...], w[0:H]) + di8[...] * t + b[...], 0.0)
    hcat = jnp.concatenate([h1r[...], h2r[...], h3], axis=1)

    bb = batr[...]
    gi = lax.broadcasted_iota(jnp.int32, (NB, G), 1)
    oh = (bb == gi).astype(jnp.float32)
    ssum[:, 0:3 * H] = ssum[:, 0:3 * H] + lax.dot_general(
        oh, hcat, (((0,), (0,)), ((), ())), preferred_element_type=jnp.float32)
    scnt[:, 0:1] = scnt[:, 0:1] + lax.dot_general(
        oh, jnp.ones((NB, 1), jnp.float32), (((0,), (0,)), ((), ())),
        preferred_element_type=jnp.float32)

    neg = jnp.float32(-jnp.inf)
    for g in range(G):
        m = bb == g
        rmax = jnp.max(jnp.where(m, hcat, neg), axis=0, keepdims=True)
        smax[g:g + 1, 0:3 * H] = jnp.maximum(smax[g:g + 1, 0:3 * H], rmax)

    @pl.when(i == NBLK - 1)
    def _():
        gmax = smax[:, 0:3 * H]
        gmax = jnp.where(gmax > jnp.float32(-3e38), gmax, 0.0)
        gmean = ssum[:, 0:3 * H] / jnp.maximum(scnt[:, 0:1], 1.0)
        pooled = jnp.concatenate([gmax, gmean], axis=1)
        out_ref[...] = _dotT(pooled, fcw[...]) + fcb[...]


_fin = pl.pallas_call(
    _fin_body,
    grid=(NBLK,),
    in_specs=[pl.BlockSpec((NB, H), lambda i: (i, 0))] * 9
    + [pl.BlockSpec((4 * H, H), lambda i: (0, 0)),
       pl.BlockSpec((1, H), lambda i: (0, 0)),
       pl.BlockSpec((NB, 1), lambda i: (i, 0)),
       pl.BlockSpec((2, 6 * H), lambda i: (0, 0)),
       pl.BlockSpec((1, 2), lambda i: (0, 0))],
    out_specs=pl.BlockSpec((G, 2), lambda i: (0, 0)),
    out_shape=_f32((G, 2)),
    scratch_shapes=[pltpu.VMEM((G, 128), jnp.float32)] * 3,
)


def kernel(x, edge_index, batch, edge_attr, conv1_w, conv1_b, conv2_w, conv2_b,
           conv3_w, conv3_b, fc_w, fc_b):
    row = edge_index[0].astype(jnp.int32)
    col = edge_index[1].astype(jnp.int32)
    ea = edge_attr.astype(jnp.float32)
    bat = batch.astype(jnp.int32)
    znN = jnp.zeros((N,), jnp.float32)
    zn8 = jnp.zeros((N, H), jnp.float32)

    d_flat, ea8 = _deg_ea8(col, ea, znN)
    d_parts = d_flat.reshape(NC, N)
    y0, z1, z2, z3, di8, dg8 = _proj(
        x, d_parts[0].reshape(N, 1), d_parts[1].reshape(N, 1),
        conv1_w.reshape(4 * H, D))

    u1_1 = _pass_first(z1, row, col, ea8, zn8)
    u1_2 = _pass_first(z2, row, col, ea8, zn8)
    u1_3 = _pass_first(z3, row, col, ea8, zn8)
    u2_2 = _pass_mid(u1_2, dg8, row, col, ea8, zn8)
    u2_3 = _pass_mid(u1_3, dg8, row, col, ea8, zn8)
    u3_3 = _pass_mid(u2_3, dg8, row, col, ea8, zn8)
    h1, g1 = _l1(y0, u1_1[0], u1_1[1], u2_2[0], u2_2[1], u3_3[0], u3_3[1],
                 di8, conv1_b.reshape(1, H))

    v1 = _pass_first(g1, row, col, ea8, zn8)
    v2 = _pass_mid(v1, dg8, row, col, ea8, zn8)
    v3 = _pass_mid(v2, dg8, row, col, ea8, zn8)
    h2, g2 = _lh(h1, v1[0], v1[1], v2[0], v2[1], v3[0], v3[1],
                 di8, conv2_w.reshape(4 * H, H), conv2_b.reshape(1, H))

    t1 = _pass_first(g2, row, col, ea8, zn8)
    t2 = _pass_mid(t1, dg8, row, col, ea8, zn8)
    t3 = _pass_mid(t2, dg8, row, col, ea8, zn8)

    out = _fin(h1, h2, t1[0], t1[1], t2[0], t2[1], t3[0], t3[1],
               di8, conv3_w.reshape(4 * H, H), conv3_b.reshape(1, H),
               bat.reshape(N, 1), fc_w, fc_b.reshape(1, 2))
    return out


# speedup vs baseline: 20.8153x; 1.1390x over previous
import functools

import jax
import jax.numpy as jnp
from jax import lax
from jax.experimental import pallas as pl
from jax.experimental.pallas import tpu as pltpu
from jax.experimental.pallas import tpu_sc as plsc

N = 10000
E = 320000
D = 128
H = 8
G = 64

NC = 2
NS = 16
NW = NC * NS
EW = E // NW
CH = 2000
NCHUNK = EW // CH
RPA = 624
TOFF = NS * RPA
TAIL = N - TOFF

NB = 1000
NBLK = N // NB

_mesh = plsc.VectorSubcoreMesh(core_axis_name="c", subcore_axis_name="s")
_sc_params = pltpu.CompilerParams(needs_layout_passes=False, use_tc_tiling_on_sc=False)


def _f32(shape):
    return jax.ShapeDtypeStruct(shape, jnp.float32)


@functools.partial(
    pl.kernel,
    out_type=[_f32((NC * N,)), _f32((E * H,))],
    mesh=_mesh,
    scratch_types=[
        pltpu.VMEM((CH,), jnp.int32),
        pltpu.VMEM((CH,), jnp.float32),
        pltpu.VMEM((CH * H,), jnp.float32),
        pltpu.VMEM((RPA,), jnp.float32),
        pltpu.VMEM_SHARED((N,), jnp.float32),
        pltpu.SemaphoreType.DMA,
    ],
    compiler_params=_sc_params,
)
def _deg_ea8(col_h, ea_h, zn_h, d_h, ea8_h, colbuf, eabuf, ea8buf, dbuf, deg_sp, sem):
    cid = lax.axis_index("c")
    sid = lax.axis_index("s")
    wid = cid * NS + sid

    @pl.when(sid == 0)
    def _():
        pltpu.sync_copy(zn_h, deg_sp)

    plsc.subcore_barrier()
    lanes = lax.iota(jnp.int32, 16)
    hi = lanes // H

    def chunk(k, carry):
        eoff = wid * EW + k * CH
        pltpu.sync_copy(col_h.at[pl.ds(eoff, CH)], colbuf)
        pltpu.sync_copy(ea_h.at[pl.ds(eoff, CH)], eabuf)

        def expand(i, c2):
            e2 = plsc.load_gather(eabuf, [hi + 2 * i])
            plsc.store_scatter(ea8buf, [lanes + 16 * i], e2)
            return c2

        lax.fori_loop(0, CH // 2, expand, 0, unroll=4)
        pltpu.sync_copy(ea8buf, ea8_h.at[pl.ds(eoff * H, CH * H)])
        pltpu.sync_copy(eabuf, deg_sp.at[colbuf], add=True)
        return carry

    lax.fori_loop(0, NCHUNK, chunk, 0)
    plsc.subcore_barrier()
    pltpu.sync_copy(deg_sp.at[pl.ds(sid * RPA, RPA)], dbuf)
    pltpu.sync_copy(dbuf, d_h.at[pl.ds(cid * N + sid * RPA, RPA)])

    @pl.when(sid == NS - 1)
    def _():
        pltpu.sync_copy(deg_sp.at[pl.ds(TOFF, TAIL)], dbuf.at[pl.ds(0, TAIL)])
        pltpu.sync_copy(dbuf.at[pl.ds(0, TAIL)],
                        d_h.at[pl.ds(cid * N + TOFF, TAIL)])


def _make_pass(two_inputs):
    scratch = (
        [pltpu.VMEM((CH,), jnp.int32)] * 3
        + [pltpu.VMEM((CH,), jnp.int32)] * 3
        + [pltpu.VMEM((CH * H,), jnp.float32)] * 3
        + [pltpu.VMEM((CH, H), jnp.float32)] * 2
        + [
            pltpu.VMEM((RPA, H), jnp.float32),
            pltpu.VMEM((RPA, H), jnp.float32),
            pltpu.VMEM((RPA, H), jnp.float32),
            pltpu.VMEM_SHARED((N, H), jnp.float32),
            pltpu.VMEM_SHARED((N, H), jnp.float32),
        ]
        + [pltpu.SemaphoreType.DMA] * 7
    )

    def body(*refs):
        if two_inputs:
            (p_h, dg8_h, row_h, col_h, ea8_h, zn8_h, q_h, *rest) = refs
        else:
            (p_h, row_h, col_h, ea8_h, zn8_h, q_h, *rest) = refs
        rowbufs = rest[0:3]
        colbufs = rest[3:6]
        eabufs = rest[6:9]
        gbufs = rest[9:11]
        pb0, pb1, dbuf, in_sp, out_sp = rest[11:16]
        ssems = rest[16:19]
        gsems = rest[19:21]
        vsems = rest[21:23]
        cid = lax.axis_index("c")
        sid = lax.axis_index("s")
        wid = cid * NS + sid
        lanes = lax.iota(jnp.int32, 16)
        ci_base = lanes // H
        jmod = lanes % H

        @pl.when(sid == 0)
        def _():
            pltpu.sync_copy(zn8_h, out_sp)
            if not two_inputs:
                pltpu.sync_copy(p_h, in_sp)

        def prologue(base, rows):
            pltpu.sync_copy(p_h.at[0, pl.ds(base, rows), :],
                            pb0.at[pl.ds(0, rows), :])
            pltpu.sync_copy(p_h.at[1, pl.ds(base, rows), :],
                            pb1.at[pl.ds(0, rows), :])
            pltpu.sync_copy(dg8_h.at[pl.ds(base, rows), :],
                            dbuf.at[pl.ds(0, rows), :])

            def comb(i, c):
                ci = ci_base + 2 * i
                v = (plsc.load_gather(pb0, [ci, jmod])
                     + plsc.load_gather(pb1, [ci, jmod]))
                v = v * plsc.load_gather(dbuf, [ci, jmod])
                plsc.store_scatter(pb0, [ci, jmod], v)
                return c

            lax.fori_loop(0, rows // 2, comb, 0, unroll=4)
            pltpu.sync_copy(pb0.at[pl.ds(0, rows), :],
                            in_sp.at[pl.ds(base, rows), :])

        if two_inputs:
            prologue(sid * RPA, RPA)

            @pl.when(sid == NS - 1)
            def _():
                prologue(TOFF, TAIL)

        plsc.subcore_barrier()

        sdesc, gdesc, vdesc = {}, {}, {}

        def stage(k):
            sl = k % 3
            eoff = wid * EW + k * CH
            sdesc[k] = [
                pltpu.async_copy(row_h.at[pl.ds(eoff, CH)], rowbufs[sl], ssems[sl]),
                pltpu.async_copy(col_h.at[pl.ds(eoff, CH)], colbufs[sl], ssems[sl]),
                pltpu.async_copy(ea8_h.at[pl.ds(eoff * H, CH * H)], eabufs[sl],
                                 ssems[sl]),
            ]

        def gather(k):
            for d in sdesc.pop(k):
                d.wait()
            gl = k % 2
            gdesc[k] = pltpu.async_copy(in_sp.at[rowbufs[k % 3]], gbufs[gl],
                                        gsems[gl])

        stage(0)
        stage(1)
        gather(0)
        for k in range(NCHUNK):
            if k + 1 < NCHUNK:
                if k >= 1:
                    vdesc.pop(k - 1).wait()
                gather(k + 1)
            gl = k % 2
            gbuf = gbufs[gl]
            eabuf = eabufs[k % 3]
            gdesc.pop(k).wait()

            def scale(i, c2):
                ci = ci_base + 2 * i
                g = plsc.load_gather(gbuf, [ci, jmod])
                e = plsc.load_gather(eabuf, [lanes + 16 * i])
                plsc.store_scatter(gbuf, [ci, jmod], g * e)
                return c2

            lax.fori_loop(0, CH // 2, scale, 0, unroll=4)
            vdesc[k] = pltpu.async_copy(gbuf, out_sp.at[colbufs[k % 3]],
                                        vsems[gl], add=True)
            if k + 2 < NCHUNK:
                stage(k + 2)
        vdesc.pop(NCHUNK - 2).wait()
        vdesc.pop(NCHUNK - 1).wait()
        plsc.subcore_barrier()
        pltpu.sync_copy(out_sp.at[pl.ds(sid * RPA, RPA), :], pb0)
        pltpu.sync_copy(pb0, q_h.at[cid, pl.ds(sid * RPA, RPA), :])

        @pl.when(sid == NS - 1)
        def _():
            pltpu.sync_copy(out_sp.at[pl.ds(TOFF, TAIL), :],
                            pb1.at[pl.ds(0, TAIL), :])
            pltpu.sync_copy(pb1.at[pl.ds(0, TAIL), :],
                            q_h.at[cid, pl.ds(TOFF, TAIL), :])

    return pl.kernel(body, out_type=_f32((NC, N, H)), mesh=_mesh,
                     scratch_types=scratch, compiler_params=_sc_params)


_pass_first = _make_pass(False)
_pass_mid = _make_pass(True)


def _proj_body(x_ref, d0_ref, d1_ref, w_ref, y0_ref, z1_ref, z2_ref, z3_ref,
               di8_ref, dg8_ref):
    deg = d0_ref[...] + d1_ref[...]
    dinv = jnp.where(deg > 0, lax.rsqrt(jnp.maximum(deg, 1e-12)), 0.0)
    dg = dinv * dinv
    yz = lax.dot_general(x_ref[...], w_ref[...], (((1,), (1,)), ((), ())),
                         preferred_element_type=jnp.float32)
    y0_ref[...] = yz[:, 0:H]
    z1_ref[...] = dinv * yz[:, H:2 * H]
    z2_ref[...] = dinv * yz[:, 2 * H:3 * H]
    z3_ref[...] = dinv * yz[:, 3 * H:4 * H]
    di8_ref[...] = jnp.broadcast_to(dinv, (NB, H))
    dg8_ref[...] = jnp.broadcast_to(dg, (NB, H))


_proj = pl.pallas_call(
    _proj_body,
    grid=(NBLK,),
    in_specs=[
        pl.BlockSpec((NB, D), lambda i: (i, 0)),
        pl.BlockSpec((NB, 1), lambda i: (i, 0)),
        pl.BlockSpec((NB, 1), lambda i: (i, 0)),
        pl.BlockSpec((4 * H, D), lambda i: (0, 0)),
    ],
    out_specs=[pl.BlockSpec((NB, H), lambda i: (i, 0))] * 6,
    out_shape=[_f32((N, H))] * 6,
)


def _l1_body(y0, u1a, u1b, u2a, u2b, u3a, u3b, di8, b, h_ref, g_ref):
    s = (u1a[...] + u1b[...]) + (u2a[...] + u2b[...]) + (u3a[...] + u3b[...])
    h = jnp.maximum(y0[...] + di8[...] * s + b[...], 0.0)
    h_ref[...] = h
    g_ref[...] = di8[...] * h


_l1 = pl.pallas_call(
    _l1_body,
    grid=(NBLK,),
    in_specs=[pl.BlockSpec((NB, H), lambda i: (i, 0))] * 8
    + [pl.BlockSpec((1, H), lambda i: (0, 0))],
    out_specs=[pl.BlockSpec((NB, H), lambda i: (i, 0))] * 2,
    out_shape=[_f32((N, H))] * 2,
)


def _dotT(a, w):
    return lax.dot_general(a, w, (((1,), (1,)), ((), ())),
                           preferred_element_type=jnp.float32)


def _lh_body(hp, v1a, v1b, v2a, v2b, v3a, v3b, di8, w_ref, b, h_ref, g_ref):
    w = w_ref[...]
    t = (_dotT(v1a[...] + v1b[...], w[H:2 * H])
         + _dotT(v2a[...] + v2b[...], w[2 * H:3 * H])
         + _dotT(v3a[...] + v3b[...], w[3 * H:4 * H]))
    h = jnp.maximum(_dotT(hp[...], w[0:H]) + di8[...] * t + b[...], 0.0)
    h_ref[...] = h
    g_ref[...] = di8[...] * h


_lh = pl.pallas_call(
    _lh_body,
    grid=(NBLK,),
    in_specs=[pl.BlockSpec((NB, H), lambda i: (i, 0))] * 8
    + [pl.BlockSpec((4 * H, H), lambda i: (0, 0)),
       pl.BlockSpec((1, H), lambda i: (0, 0))],
    out_specs=[pl.BlockSpec((NB, H), lambda i: (i, 0))] * 2,
    out_shape=[_f32((N, H))] * 2,
)


def _fin_body(h1r, h2r, v1a, v1b, v2a, v2b, v3a, v3b, di8, w_ref, b, batr,
              fcw, fcb, out_ref, smax, ssum, scnt):
    i = pl.program_id(0)

    @pl.when(i == 0)
    def _():
        smax[...] = jnp.full((G, 128), -jnp.inf, jnp.float32)
        ssum[...] = jnp.zeros((G, 128), jnp.float32)
        scnt[...] = jnp.zeros((G, 128), jnp.float32)

    w = w_ref[...]
    t = (_dotT(v1a[...] + v1b[...], w[H:2 * H])
         + _dotT(v2a[...] + v2b[...], w[2 * H:3 * H])
         + _dotT(v3a[...] + v3b[...], w[3 * H:4 * H]))
    h3 = jnp.maximum(_dotT(h2r[...], w[0:H]) + di8[...] * t + b[...], 0.0)
    hcat = jnp.concatenate([h1r[...], h2r[...], h3], axis=1)

    bb = batr[...]
    gi = lax.broadcasted_iota(jnp.int32, (NB, G), 1)
    oh = (bb == gi).astype(jnp.float32)
    ssum[:, 0:3 * H] = ssum[:, 0:3 * H] + lax.dot_general(
        oh, hcat, (((0,), (0,)), ((), ())), preferred_element_type=jnp.float32)
    scnt[:, 0:1] = scnt[:, 0:1] + lax.dot_general(
        oh, jnp.ones((NB, 1), jnp.float32), (((0,), (0,)), ((), ())),
        preferred_element_type=jnp.float32)

    neg = jnp.float32(-jnp.inf)
    for g in range(G):
        m = bb == g
        rmax = jnp.max(jnp.where(m, hcat, neg), axis=0, keepdims=True)
        smax[g:g + 1, 0:3 * H] = jnp.maximum(smax[g:g + 1, 0:3 * H], rmax)

    @pl.when(i == NBLK - 1)
    def _():
        gmax = smax[:, 0:3 * H]
        gmax = jnp.where(gmax > jnp.float32(-3e38), gmax, 0.0)
        gmean = ssum[:, 0:3 * H] / jnp.maximum(scnt[:, 0:1], 1.0)
        pooled = jnp.concatenate([gmax, gmean], axis=1)
        out_ref[...] = _dotT(pooled, fcw[...]) + fcb[...]


_fin = pl.pallas_call(
    _fin_body,
    grid=(NBLK,),
    in_specs=[pl.BlockSpec((NB, H), lambda i: (i, 0))] * 9
    + [pl.BlockSpec((4 * H, H), lambda i: (0, 0)),
       pl.BlockSpec((1, H), lambda i: (0, 0)),
       pl.BlockSpec((NB, 1), lambda i: (i, 0)),
       pl.BlockSpec((2, 6 * H), lambda i: (0, 0)),
       pl.BlockSpec((1, 2), lambda i: (0, 0))],
    out_specs=pl.BlockSpec((G, 2), lambda i: (0, 0)),
    out_shape=_f32((G, 2)),
    scratch_shapes=[pltpu.VMEM((G, 128), jnp.float32)] * 3,
)


def kernel(x, edge_index, batch, edge_attr, conv1_w, conv1_b, conv2_w, conv2_b,
           conv3_w, conv3_b, fc_w, fc_b):
    row = edge_index[0].astype(jnp.int32)
    col = edge_index[1].astype(jnp.int32)
    ea = edge_attr.astype(jnp.float32)
    bat = batch.astype(jnp.int32)
    znN = jnp.zeros((N,), jnp.float32)
    zn8 = jnp.zeros((N, H), jnp.float32)

    d_flat, ea8 = _deg_ea8(col, ea, znN)
    d_parts = d_flat.reshape(NC, N)
    y0, z1, z2, z3, di8, dg8 = _proj(
        x, d_parts[0].reshape(N, 1), d_parts[1].reshape(N, 1),
        conv1_w.reshape(4 * H, D))

    u1_1 = _pass_first(z1, row, col, ea8, zn8)
    u1_2 = _pass_first(z2, row, col, ea8, zn8)
    u1_3 = _pass_first(z3, row, col, ea8, zn8)
    u2_2 = _pass_mid(u1_2, dg8, row, col, ea8, zn8)
    u2_3 = _pass_mid(u1_3, dg8, row, col, ea8, zn8)
    u3_3 = _pass_mid(u2_3, dg8, row, col, ea8, zn8)
    h1, g1 = _l1(y0, u1_1[0], u1_1[1], u2_2[0], u2_2[1], u3_3[0], u3_3[1],
                 di8, conv1_b.reshape(1, H))

    v1 = _pass_first(g1, row, col, ea8, zn8)
    v2 = _pass_mid(v1, dg8, row, col, ea8, zn8)
    v3 = _pass_mid(v2, dg8, row, col, ea8, zn8)
    h2, g2 = _lh(h1, v1[0], v1[1], v2[0], v2[1], v3[0], v3[1],
                 di8, conv2_w.reshape(4 * H, H), conv2_b.reshape(1, H))

    t1 = _pass_first(g2, row, col, ea8, zn8)
    t2 = _pass_mid(t1, dg8, row, col, ea8, zn8)
    t3 = _pass_mid(t2, dg8, row, col, ea8, zn8)

    out = _fin(h1, h2, t1[0], t1[1], t2[0], t2[1], t3[0], t3[1],
               di8, conv3_w.reshape(4 * H, H), conv3_b.reshape(1, H),
               bat.reshape(N, 1), fc_w, fc_b.reshape(1, 2))
    return out


# speedup vs baseline: 29.1312x; 1.3995x over previous
import functools

import jax
import jax.numpy as jnp
from jax import lax
from jax.experimental import pallas as pl
from jax.experimental.pallas import tpu as pltpu
from jax.experimental.pallas import tpu_sc as plsc

N = 10000
E = 320000
D = 128
H = 8
G = 64

NC = 2
NS = 16
NW = NC * NS
EW = E // NW
CH = 2000
NCHUNK = EW // CH
RPA = 624
TOFF = NS * RPA
TAIL = N - TOFF

NB = 1000
NBLK = N // NB

_mesh = plsc.VectorSubcoreMesh(core_axis_name="c", subcore_axis_name="s")
_sc_params = pltpu.CompilerParams(needs_layout_passes=False, use_tc_tiling_on_sc=False)


def _f32(shape):
    return jax.ShapeDtypeStruct(shape, jnp.float32)


@functools.partial(
    pl.kernel,
    out_type=[_f32((NC * N,)), _f32((E * H,))],
    mesh=_mesh,
    scratch_types=[
        pltpu.VMEM((CH,), jnp.int32),
        pltpu.VMEM((CH,), jnp.float32),
        pltpu.VMEM((CH * H,), jnp.float32),
        pltpu.VMEM((RPA,), jnp.float32),
        pltpu.VMEM_SHARED((N,), jnp.float32),
        pltpu.SemaphoreType.DMA,
    ],
    compiler_params=_sc_params,
)
def _deg_ea8(col_h, ea_h, zn_h, d_h, ea8_h, colbuf, eabuf, ea8buf, dbuf, deg_sp, sem):
    cid = lax.axis_index("c")
    sid = lax.axis_index("s")
    wid = cid * NS + sid

    @pl.when(sid == 0)
    def _():
        pltpu.sync_copy(zn_h, deg_sp)

    plsc.subcore_barrier()
    lanes = lax.iota(jnp.int32, 16)
    hi = lanes // H

    def chunk(k, carry):
        eoff = wid * EW + k * CH
        pltpu.sync_copy(col_h.at[pl.ds(eoff, CH)], colbuf)
        pltpu.sync_copy(ea_h.at[pl.ds(eoff, CH)], eabuf)

        def expand(i, c2):
            e2 = plsc.load_gather(eabuf, [hi + 2 * i])
            plsc.store_scatter(ea8buf, [lanes + 16 * i], e2)
            return c2

        lax.fori_loop(0, CH // 2, expand, 0, unroll=4)
        pltpu.sync_copy(ea8buf, ea8_h.at[pl.ds(eoff * H, CH * H)])
        pltpu.sync_copy(eabuf, deg_sp.at[colbuf], add=True)
        return carry

    lax.fori_loop(0, NCHUNK, chunk, 0)
    plsc.subcore_barrier()
    pltpu.sync_copy(deg_sp.at[pl.ds(sid * RPA, RPA)], dbuf)
    pltpu.sync_copy(dbuf, d_h.at[pl.ds(cid * N + sid * RPA, RPA)])

    @pl.when(sid == NS - 1)
    def _():
        pltpu.sync_copy(deg_sp.at[pl.ds(TOFF, TAIL)], dbuf.at[pl.ds(0, TAIL)])
        pltpu.sync_copy(dbuf.at[pl.ds(0, TAIL)],
                        d_h.at[pl.ds(cid * N + TOFF, TAIL)])


def _make_pass(two_inputs):
    scratch = (
        [pltpu.VMEM((CH,), jnp.int32)] * 3
        + [pltpu.VMEM((CH,), jnp.int32)] * 3
        + [pltpu.VMEM((CH * H,), jnp.float32)] * 3
        + [pltpu.VMEM((CH, H), jnp.float32)] * 2
        + [
            pltpu.VMEM((RPA, H), jnp.float32),
            pltpu.VMEM((RPA, H), jnp.float32),
            pltpu.VMEM((RPA, H), jnp.float32),
            pltpu.VMEM_SHARED((N, H), jnp.float32),
            pltpu.VMEM_SHARED((N, H), jnp.float32),
        ]
        + [pltpu.SemaphoreType.DMA] * 7
    )

    def body(*refs):
        if two_inputs:
            (p_h, dg8_h, row_h, col_h, ea8_h, zn8_h, q_h, *rest) = refs
        else:
            (p_h, row_h, col_h, ea8_h, zn8_h, q_h, *rest) = refs
        rowbufs = rest[0:3]
        colbufs = rest[3:6]
        eabufs = rest[6:9]
        gbufs = rest[9:11]
        pb0, pb1, dbuf, in_sp, out_sp = rest[11:16]
        ssems = rest[16:19]
        gsems = rest[19:21]
        vsems = rest[21:23]
        cid = lax.axis_index("c")
        sid = lax.axis_index("s")
        wid = cid * NS + sid
        lanes = lax.iota(jnp.int32, 16)
        ci_base = lanes // H
        jmod = lanes % H

        @pl.when(sid == 0)
        def _():
            pltpu.sync_copy(zn8_h, out_sp)
            if not two_inputs:
                pltpu.sync_copy(p_h, in_sp)

        def prologue(base, rows):
            pltpu.sync_copy(p_h.at[0, pl.ds(base, rows), :],
                            pb0.at[pl.ds(0, rows), :])
            pltpu.sync_copy(p_h.at[1, pl.ds(base, rows), :],
                            pb1.at[pl.ds(0, rows), :])
            pltpu.sync_copy(dg8_h.at[pl.ds(base, rows), :],
                            dbuf.at[pl.ds(0, rows), :])

            def comb(i, c):
                ci = ci_base + 2 * i
                v = (plsc.load_gather(pb0, [ci, jmod])
                     + plsc.load_gather(pb1, [ci, jmod]))
                v = v * plsc.load_gather(dbuf, [ci, jmod])
                plsc.store_scatter(pb0, [ci, jmod], v)
                return c

            lax.fori_loop(0, rows // 2, comb, 0, unroll=4)
            pltpu.sync_copy(pb0.at[pl.ds(0, rows), :],
                            in_sp.at[pl.ds(base, rows), :])

        if two_inputs:
            prologue(sid * RPA, RPA)

            @pl.when(sid == NS - 1)
            def _():
                prologue(TOFF, TAIL)

        plsc.subcore_barrier()

        sdesc, gdesc, vdesc = {}, {}, {}

        def stage(k):
            sl = k % 3
            eoff = wid * EW + k * CH
            sdesc[k] = [
                pltpu.async_copy(row_h.at[pl.ds(eoff, CH)], rowbufs[sl], ssems[sl]),
                pltpu.async_copy(col_h.at[pl.ds(eoff, CH)], colbufs[sl], ssems[sl]),
                pltpu.async_copy(ea8_h.at[pl.ds(eoff * H, CH * H)], eabufs[sl],
                                 ssems[sl]),
            ]

        def gather(k):
            for d in sdesc.pop(k):
                d.wait()
            gl = k % 2
            gdesc[k] = pltpu.async_copy(in_sp.at[rowbufs[k % 3]], gbufs[gl],
                                        gsems[gl])

        stage(0)
        stage(1)
        gather(0)
        for k in range(NCHUNK):
            if k + 1 < NCHUNK:
                if k >= 1:
                    vdesc.pop(k - 1).wait()
                gather(k + 1)
            gl = k % 2
            gbuf = gbufs[gl]
            eabuf = eabufs[k % 3]
            gdesc.pop(k).wait()

            @plsc.parallel_loop(0, CH // 2, unroll=8)
            def _(i):
                ci = ci_base + 2 * i
                g = plsc.load_gather(gbuf, [ci, jmod])
                e = plsc.load_gather(eabuf, [lanes + 16 * i])
                plsc.store_scatter(gbuf, [ci, jmod], g * e)
            vdesc[k] = pltpu.async_copy(gbuf, out_sp.at[colbufs[k % 3]],
                                        vsems[gl], add=True)
            if k + 2 < NCHUNK:
                stage(k + 2)
        vdesc.pop(NCHUNK - 2).wait()
        vdesc.pop(NCHUNK - 1).wait()
        plsc.subcore_barrier()
        pltpu.sync_copy(out_sp.at[pl.ds(sid * RPA, RPA), :], pb0)
        pltpu.sync_copy(pb0, q_h.at[cid, pl.ds(sid * RPA, RPA), :])

        @pl.when(sid == NS - 1)
        def _():
            pltpu.sync_copy(out_sp.at[pl.ds(TOFF, TAIL), :],
                            pb1.at[pl.ds(0, TAIL), :])
            pltpu.sync_copy(pb1.at[pl.ds(0, TAIL), :],
                            q_h.at[cid, pl.ds(TOFF, TAIL), :])

    return pl.kernel(body, out_type=_f32((NC, N, H)), mesh=_mesh,
                     scratch_types=scratch, compiler_params=_sc_params)


_pass_first = _make_pass(False)
_pass_mid = _make_pass(True)


def _proj_body(x_ref, d0_ref, d1_ref, w_ref, y0_ref, z1_ref, z2_ref, z3_ref,
               di8_ref, dg8_ref):
    deg = d0_ref[...] + d1_ref[...]
    dinv = jnp.where(deg > 0, lax.rsqrt(jnp.maximum(deg, 1e-12)), 0.0)
    dg = dinv * dinv
    yz = lax.dot_general(x_ref[...], w_ref[...], (((1,), (1,)), ((), ())),
                         preferred_element_type=jnp.float32)
    y0_ref[...] = yz[:, 0:H]
    z1_ref[...] = dinv * yz[:, H:2 * H]
    z2_ref[...] = dinv * yz[:, 2 * H:3 * H]
    z3_ref[...] = dinv * yz[:, 3 * H:4 * H]
    di8_ref[...] = jnp.broadcast_to(dinv, (NB, H))
    dg8_ref[...] = jnp.broadcast_to(dg, (NB, H))


_proj = pl.pallas_call(
    _proj_body,
    grid=(NBLK,),
    in_specs=[
        pl.BlockSpec((NB, D), lambda i: (i, 0)),
        pl.BlockSpec((NB, 1), lambda i: (i, 0)),
        pl.BlockSpec((NB, 1), lambda i: (i, 0)),
        pl.BlockSpec((4 * H, D), lambda i: (0, 0)),
    ],
    out_specs=[pl.BlockSpec((NB, H), lambda i: (i, 0))] * 6,
    out_shape=[_f32((N, H))] * 6,
)


def _l1_body(y0, u1a, u1b, u2a, u2b, u3a, u3b, di8, b, h_ref, g_ref):
    s = (u1a[...] + u1b[...]) + (u2a[...] + u2b[...]) + (u3a[...] + u3b[...])
    h = jnp.maximum(y0[...] + di8[...] * s + b[...], 0.0)
    h_ref[...] = h
    g_ref[...] = di8[...] * h


_l1 = pl.pallas_call(
    _l1_body,
    grid=(NBLK,),
    in_specs=[pl.BlockSpec((NB, H), lambda i: (i, 0))] * 8
    + [pl.BlockSpec((1, H), lambda i: (0, 0))],
    out_specs=[pl.BlockSpec((NB, H), lambda i: (i, 0))] * 2,
    out_shape=[_f32((N, H))] * 2,
)


def _dotT(a, w):
    return lax.dot_general(a, w, (((1,), (1,)), ((), ())),
                           preferred_element_type=jnp.float32)


def _lh_body(hp, v1a, v1b, v2a, v2b, v3a, v3b, di8, w_ref, b, h_ref, g_ref):
    w = w_ref[...]
    t = (_dotT(v1a[...] + v1b[...], w[H:2 * H])
         + _dotT(v2a[...] + v2b[...], w[2 * H:3 * H])
         + _dotT(v3a[...] + v3b[...], w[3 * H:4 * H]))
    h = jnp.maximum(_dotT(hp[...], w[0:H]) + di8[...] * t + b[...], 0.0)
    h_ref[...] = h
    g_ref[...] = di8[...] * h


_lh = pl.pallas_call(
    _lh_body,
    grid=(NBLK,),
    in_specs=[pl.BlockSpec((NB, H), lambda i: (i, 0))] * 8
    + [pl.BlockSpec((4 * H, H), lambda i: (0, 0)),
       pl.BlockSpec((1, H), lambda i: (0, 0))],
    out_specs=[pl.BlockSpec((NB, H), lambda i: (i, 0))] * 2,
    out_shape=[_f32((N, H))] * 2,
)


def _fin_body(h1r, h2r, v1a, v1b, v2a, v2b, v3a, v3b, di8, w_ref, b, batr,
              fcw, fcb, out_ref, smax, ssum, scnt):
    i = pl.program_id(0)

    @pl.when(i == 0)
    def _():
        smax[...] = jnp.full((G, 128), -jnp.inf, jnp.float32)
        ssum[...] = jnp.zeros((G, 128), jnp.float32)
        scnt[...] = jnp.zeros((G, 128), jnp.float32)

    w = w_ref[...]
    t = (_dotT(v1a[...] + v1b[...], w[H:2 * H])
         + _dotT(v2a[...] + v2b[...], w[2 * H:3 * H])
         + _dotT(v3a[...] + v3b[...], w[3 * H:4 * H]))
    h3 = jnp.maximum(_dotT(h2r[...], w[0:H]) + di8[...] * t + b[...], 0.0)
    hcat = jnp.concatenate([h1r[...], h2r[...], h3], axis=1)

    bb = batr[...]
    gi = lax.broadcasted_iota(jnp.int32, (NB, G), 1)
    oh = (bb == gi).astype(jnp.float32)
    ssum[:, 0:3 * H] = ssum[:, 0:3 * H] + lax.dot_general(
        oh, hcat, (((0,), (0,)), ((), ())), preferred_element_type=jnp.float32)
    scnt[:, 0:1] = scnt[:, 0:1] + lax.dot_general(
        oh, jnp.ones((NB, 1), jnp.float32), (((0,), (0,)), ((), ())),
        preferred_element_type=jnp.float32)

    neg = jnp.float32(-jnp.inf)
    for g in range(G):
        m = bb == g
        rmax = jnp.max(jnp.where(m, hcat, neg), axis=0, keepdims=True)
        smax[g:g + 1, 0:3 * H] = jnp.maximum(smax[g:g + 1, 0:3 * H], rmax)

    @pl.when(i == NBLK - 1)
    def _():
        gmax = smax[:, 0:3 * H]
        gmax = jnp.where(gmax > jnp.float32(-3e38), gmax, 0.0)
        gmean = ssum[:, 0:3 * H] / jnp.maximum(scnt[:, 0:1], 1.0)
        pooled = jnp.concatenate([gmax, gmean], axis=1)
        out_ref[...] = _dotT(pooled, fcw[...]) + fcb[...]


_fin = pl.pallas_call(
    _fin_body,
    grid=(NBLK,),
    in_specs=[pl.BlockSpec((NB, H), lambda i: (i, 0))] * 9
    + [pl.BlockSpec((4 * H, H), lambda i: (0, 0)),
       pl.BlockSpec((1, H), lambda i: (0, 0)),
       pl.BlockSpec((NB, 1), lambda i: (i, 0)),
       pl.BlockSpec((2, 6 * H), lambda i: (0, 0)),
       pl.BlockSpec((1, 2), lambda i: (0, 0))],
    out_specs=pl.BlockSpec((G, 2), lambda i: (0, 0)),
    out_shape=_f32((G, 2)),
    scratch_shapes=[pltpu.VMEM((G, 128), jnp.float32)] * 3,
)


def kernel(x, edge_index, batch, edge_attr, conv1_w, conv1_b, conv2_w, conv2_b,
           conv3_w, conv3_b, fc_w, fc_b):
    row = edge_index[0].astype(jnp.int32)
    col = edge_index[1].astype(jnp.int32)
    ea = edge_attr.astype(jnp.float32)
    bat = batch.astype(jnp.int32)
    znN = jnp.zeros((N,), jnp.float32)
    zn8 = jnp.zeros((N, H), jnp.float32)

    d_flat, ea8 = _deg_ea8(col, ea, znN)
    d_parts = d_flat.reshape(NC, N)
    y0, z1, z2, z3, di8, dg8 = _proj(
        x, d_parts[0].reshape(N, 1), d_parts[1].reshape(N, 1),
        conv1_w.reshape(4 * H, D))

    u1_1 = _pass_first(z1, row, col, ea8, zn8)
    u1_2 = _pass_first(z2, row, col, ea8, zn8)
    u1_3 = _pass_first(z3, row, col, ea8, zn8)
    u2_2 = _pass_mid(u1_2, dg8, row, col, ea8, zn8)
    u2_3 = _pass_mid(u1_3, dg8, row, col, ea8, zn8)
    u3_3 = _pass_mid(u2_3, dg8, row, col, ea8, zn8)
    h1, g1 = _l1(y0, u1_1[0], u1_1[1], u2_2[0], u2_2[1], u3_3[0], u3_3[1],
                 di8, conv1_b.reshape(1, H))

    v1 = _pass_first(g1, row, col, ea8, zn8)
    v2 = _pass_mid(v1, dg8, row, col, ea8, zn8)
    v3 = _pass_mid(v2, dg8, row, col, ea8, zn8)
    h2, g2 = _lh(h1, v1[0], v1[1], v2[0], v2[1], v3[0], v3[1],
                 di8, conv2_w.reshape(4 * H, H), conv2_b.reshape(1, H))

    t1 = _pass_first(g2, row, col, ea8, zn8)
    t2 = _pass_mid(t1, dg8, row, col, ea8, zn8)
    t3 = _pass_mid(t2, dg8, row, col, ea8, zn8)

    out = _fin(h1, h2, t1[0], t1[1], t2[0], t2[1], t3[0], t3[1],
               di8, conv3_w.reshape(4 * H, H), conv3_b.reshape(1, H),
               bat.reshape(N, 1), fc_w, fc_b.reshape(1, 2))
    return out


# speedup vs baseline: 30.2228x; 1.0375x over previous
import functools

import jax
import jax.numpy as jnp
from jax import lax
from jax.experimental import pallas as pl
from jax.experimental.pallas import tpu as pltpu
from jax.experimental.pallas import tpu_sc as plsc

N = 10000
E = 320000
D = 128
H = 8
G = 64

NC = 2
NS = 16
NW = NC * NS
EW = E // NW
CH = 2000
NCHUNK = EW // CH
RPA = 624
TOFF = NS * RPA
TAIL = N - TOFF

NB = 1000
NBLK = N // NB

_mesh = plsc.VectorSubcoreMesh(core_axis_name="c", subcore_axis_name="s")
_sc_params = pltpu.CompilerParams(needs_layout_passes=False, use_tc_tiling_on_sc=False)


def _f32(shape):
    return jax.ShapeDtypeStruct(shape, jnp.float32)


@functools.partial(
    pl.kernel,
    out_type=[_f32((NC * N,)), _f32((E * H,))],
    mesh=_mesh,
    scratch_types=[
        pltpu.VMEM((CH,), jnp.int32),
        pltpu.VMEM((CH,), jnp.float32),
        pltpu.VMEM((CH * H,), jnp.float32),
        pltpu.VMEM((RPA,), jnp.float32),
        pltpu.VMEM_SHARED((N,), jnp.float32),
        pltpu.SemaphoreType.DMA,
    ],
    compiler_params=_sc_params,
)
def _deg_ea8(col_h, ea_h, zn_h, d_h, ea8_h, colbuf, eabuf, ea8buf, dbuf, deg_sp, sem):
    cid = lax.axis_index("c")
    sid = lax.axis_index("s")
    wid = cid * NS + sid

    @pl.when(sid == 0)
    def _():
        pltpu.sync_copy(zn_h, deg_sp)

    plsc.subcore_barrier()
    lanes = lax.iota(jnp.int32, 16)
    hi = lanes // H

    def chunk(k, carry):
        eoff = wid * EW + k * CH
        pltpu.sync_copy(col_h.at[pl.ds(eoff, CH)], colbuf)
        pltpu.sync_copy(ea_h.at[pl.ds(eoff, CH)], eabuf)

        @plsc.parallel_loop(0, CH // 2, unroll=8)
        def _(i):
            e2 = plsc.load_gather(eabuf, [hi + 2 * i])
            plsc.store_scatter(ea8buf, [lanes + 16 * i], e2)
        pltpu.sync_copy(ea8buf, ea8_h.at[pl.ds(eoff * H, CH * H)])
        pltpu.sync_copy(eabuf, deg_sp.at[colbuf], add=True)
        return carry

    lax.fori_loop(0, NCHUNK, chunk, 0)
    plsc.subcore_barrier()
    pltpu.sync_copy(deg_sp.at[pl.ds(sid * RPA, RPA)], dbuf)
    pltpu.sync_copy(dbuf, d_h.at[pl.ds(cid * N + sid * RPA, RPA)])

    @pl.when(sid == NS - 1)
    def _():
        pltpu.sync_copy(deg_sp.at[pl.ds(TOFF, TAIL)], dbuf.at[pl.ds(0, TAIL)])
        pltpu.sync_copy(dbuf.at[pl.ds(0, TAIL)],
                        d_h.at[pl.ds(cid * N + TOFF, TAIL)])


def _make_pass(two_inputs):
    scratch = (
        [pltpu.VMEM((CH,), jnp.int32)] * 3
        + [pltpu.VMEM((CH,), jnp.int32)] * 3
        + [pltpu.VMEM((CH * H,), jnp.float32)] * 3
        + [pltpu.VMEM((CH, H), jnp.float32)] * 2
        + [
            pltpu.VMEM((RPA, H), jnp.float32),
            pltpu.VMEM((RPA, H), jnp.float32),
            pltpu.VMEM((RPA, H), jnp.float32),
            pltpu.VMEM_SHARED((N, H), jnp.float32),
            pltpu.VMEM_SHARED((N, H), jnp.float32),
        ]
        + [pltpu.SemaphoreType.DMA] * 7
    )

    def body(*refs):
        if two_inputs:
            (p_h, dg8_h, row_h, col_h, ea8_h, zn8_h, q_h, *rest) = refs
        else:
            (p_h, row_h, col_h, ea8_h, zn8_h, q_h, *rest) = refs
        rowbufs = rest[0:3]
        colbufs = rest[3:6]
        eabufs = rest[6:9]
        gbufs = rest[9:11]
        pb0, pb1, dbuf, in_sp, out_sp = rest[11:16]
        ssems = rest[16:19]
        gsems = rest[19:21]
        vsems = rest[21:23]
        cid = lax.axis_index("c")
        sid = lax.axis_index("s")
        wid = cid * NS + sid
        lanes = lax.iota(jnp.int32, 16)
        ci_base = lanes // H
        jmod = lanes % H

        @pl.when(sid == 0)
        def _():
            pltpu.sync_copy(zn8_h, out_sp)
            if not two_inputs:
                pltpu.sync_copy(p_h, in_sp)

        def prologue(base, rows):
            pltpu.sync_copy(p_h.at[0, pl.ds(base, rows), :],
                            pb0.at[pl.ds(0, rows), :])
            pltpu.sync_copy(p_h.at[1, pl.ds(base, rows), :],
                            pb1.at[pl.ds(0, rows), :])
            pltpu.sync_copy(dg8_h.at[pl.ds(base, rows), :],
                            dbuf.at[pl.ds(0, rows), :])

            @plsc.parallel_loop(0, rows // 2, unroll=8)
            def _(i):
                ci = ci_base + 2 * i
                v = (plsc.load_gather(pb0, [ci, jmod])
                     + plsc.load_gather(pb1, [ci, jmod]))
                v = v * plsc.load_gather(dbuf, [ci, jmod])
                plsc.store_scatter(pb0, [ci, jmod], v)
            pltpu.sync_copy(pb0.at[pl.ds(0, rows), :],
                            in_sp.at[pl.ds(base, rows), :])

        if two_inputs:
            prologue(sid * RPA, RPA)

            @pl.when(sid == NS - 1)
            def _():
                prologue(TOFF, TAIL)

        plsc.subcore_barrier()

        sdesc, gdesc, vdesc = {}, {}, {}

        def stage(k):
            sl = k % 3
            eoff = wid * EW + k * CH
            sdesc[k] = [
                pltpu.async_copy(row_h.at[pl.ds(eoff, CH)], rowbufs[sl], ssems[sl]),
                pltpu.async_copy(col_h.at[pl.ds(eoff, CH)], colbufs[sl], ssems[sl]),
                pltpu.async_copy(ea8_h.at[pl.ds(eoff * H, CH * H)], eabufs[sl],
                                 ssems[sl]),
            ]

        def gather(k):
            for d in sdesc.pop(k):
                d.wait()
            gl = k % 2
            gdesc[k] = pltpu.async_copy(in_sp.at[rowbufs[k % 3]], gbufs[gl],
                                        gsems[gl])

        stage(0)
        stage(1)
        gather(0)
        for k in range(NCHUNK):
            if k + 1 < NCHUNK:
                if k >= 1:
                    vdesc.pop(k - 1).wait()
                gather(k + 1)
            gl = k % 2
            gbuf = gbufs[gl]
            eabuf = eabufs[k % 3]
            gdesc.pop(k).wait()

            @plsc.parallel_loop(0, CH // 2, unroll=8)
            def _(i):
                ci = ci_base + 2 * i
                g = plsc.load_gather(gbuf, [ci, jmod])
                e = plsc.load_gather(eabuf, [lanes + 16 * i])
                plsc.store_scatter(gbuf, [ci, jmod], g * e)
            vdesc[k] = pltpu.async_copy(gbuf, out_sp.at[colbufs[k % 3]],
                                        vsems[gl], add=True)
            if k + 2 < NCHUNK:
                stage(k + 2)
        vdesc.pop(NCHUNK - 2).wait()
        vdesc.pop(NCHUNK - 1).wait()
        plsc.subcore_barrier()
        pltpu.sync_copy(out_sp.at[pl.ds(sid * RPA, RPA), :], pb0)
        pltpu.sync_copy(pb0, q_h.at[cid, pl.ds(sid * RPA, RPA), :])

        @pl.when(sid == NS - 1)
        def _():
            pltpu.sync_copy(out_sp.at[pl.ds(TOFF, TAIL), :],
                            pb1.at[pl.ds(0, TAIL), :])
            pltpu.sync_copy(pb1.at[pl.ds(0, TAIL), :],
                            q_h.at[cid, pl.ds(TOFF, TAIL), :])

    return pl.kernel(body, out_type=_f32((NC, N, H)), mesh=_mesh,
                     scratch_types=scratch, compiler_params=_sc_params)


_pass_first = _make_pass(False)
_pass_mid = _make_pass(True)


def _proj_body(x_ref, d0_ref, d1_ref, w_ref, y0_ref, z1_ref, z2_ref, z3_ref,
               di8_ref, dg8_ref):
    deg = d0_ref[...] + d1_ref[...]
    dinv = jnp.where(deg > 0, lax.rsqrt(jnp.maximum(deg, 1e-12)), 0.0)
    dg = dinv * dinv
    yz = lax.dot_general(x_ref[...], w_ref[...], (((1,), (1,)), ((), ())),
                         preferred_element_type=jnp.float32)
    y0_ref[...] = yz[:, 0:H]
    z1_ref[...] = dinv * yz[:, H:2 * H]
    z2_ref[...] = dinv * yz[:, 2 * H:3 * H]
    z3_ref[...] = dinv * yz[:, 3 * H:4 * H]
    di8_ref[...] = jnp.broadcast_to(dinv, (NB, H))
    dg8_ref[...] = jnp.broadcast_to(dg, (NB, H))


_proj = pl.pallas_call(
    _proj_body,
    grid=(NBLK,),
    in_specs=[
        pl.BlockSpec((NB, D), lambda i: (i, 0)),
        pl.BlockSpec((NB, 1), lambda i: (i, 0)),
        pl.BlockSpec((NB, 1), lambda i: (i, 0)),
        pl.BlockSpec((4 * H, D), lambda i: (0, 0)),
    ],
    out_specs=[pl.BlockSpec((NB, H), lambda i: (i, 0))] * 6,
    out_shape=[_f32((N, H))] * 6,
)


def _l1_body(y0, u1a, u1b, u2a, u2b, u3a, u3b, di8, b, h_ref, g_ref):
    s = (u1a[...] + u1b[...]) + (u2a[...] + u2b[...]) + (u3a[...] + u3b[...])
    h = jnp.maximum(y0[...] + di8[...] * s + b[...], 0.0)
    h_ref[...] = h
    g_ref[...] = di8[...] * h


_l1 = pl.pallas_call(
    _l1_body,
    grid=(NBLK,),
    in_specs=[pl.BlockSpec((NB, H), lambda i: (i, 0))] * 8
    + [pl.BlockSpec((1, H), lambda i: (0, 0))],
    out_specs=[pl.BlockSpec((NB, H), lambda i: (i, 0))] * 2,
    out_shape=[_f32((N, H))] * 2,
)


def _dotT(a, w):
    return lax.dot_general(a, w, (((1,), (1,)), ((), ())),
                           preferred_element_type=jnp.float32)


def _lh_body(hp, v1a, v1b, v2a, v2b, v3a, v3b, di8, w_ref, b, h_ref, g_ref):
    w = w_ref[...]
    t = (_dotT(v1a[...] + v1b[...], w[H:2 * H])
         + _dotT(v2a[...] + v2b[...], w[2 * H:3 * H])
         + _dotT(v3a[...] + v3b[...], w[3 * H:4 * H]))
    h = jnp.maximum(_dotT(hp[...], w[0:H]) + di8[...] * t + b[...], 0.0)
    h_ref[...] = h
    g_ref[...] = di8[...] * h


_lh = pl.pallas_call(
    _lh_body,
    grid=(NBLK,),
    in_specs=[pl.BlockSpec((NB, H), lambda i: (i, 0))] * 8
    + [pl.BlockSpec((4 * H, H), lambda i: (0, 0)),
       pl.BlockSpec((1, H), lambda i: (0, 0))],
    out_specs=[pl.BlockSpec((NB, H), lambda i: (i, 0))] * 2,
    out_shape=[_f32((N, H))] * 2,
)


def _fin_body(h1r, h2r, v1a, v1b, v2a, v2b, v3a, v3b, di8, w_ref, b, batr,
              fcw, fcb, out_ref, smax, ssum, scnt):
    i = pl.program_id(0)

    @pl.when(i == 0)
    def _():
        smax[...] = jnp.full((G, 128), -jnp.inf, jnp.float32)
        ssum[...] = jnp.zeros((G, 128), jnp.float32)
        scnt[...] = jnp.zeros((G, 128), jnp.float32)

    w = w_ref[...]
    t = (_dotT(v1a[...] + v1b[...], w[H:2 * H])
         + _dotT(v2a[...] + v2b[...], w[2 * H:3 * H])
         + _dotT(v3a[...] + v3b[...], w[3 * H:4 * H]))
    h3 = jnp.maximum(_dotT(h2r[...], w[0:H]) + di8[...] * t + b[...], 0.0)
    hcat = jnp.concatenate([h1r[...], h2r[...], h3], axis=1)

    bb = batr[...]
    gi = lax.broadcasted_iota(jnp.int32, (NB, G), 1)
    oh = (bb == gi).astype(jnp.float32)
    ssum[:, 0:3 * H] = ssum[:, 0:3 * H] + lax.dot_general(
        oh, hcat, (((0,), (0,)), ((), ())), preferred_element_type=jnp.float32)
    scnt[:, 0:1] = scnt[:, 0:1] + lax.dot_general(
        oh, jnp.ones((NB, 1), jnp.float32), (((0,), (0,)), ((), ())),
        preferred_element_type=jnp.float32)

    neg = jnp.float32(-jnp.inf)
    for g in range(G):
        m = bb == g
        rmax = jnp.max(jnp.where(m, hcat, neg), axis=0, keepdims=True)
        smax[g:g + 1, 0:3 * H] = jnp.maximum(smax[g:g + 1, 0:3 * H], rmax)

    @pl.when(i == NBLK - 1)
    def _():
        gmax = smax[:, 0:3 * H]
        gmax = jnp.where(gmax > jnp.float32(-3e38), gmax, 0.0)
        gmean = ssum[:, 0:3 * H] / jnp.maximum(scnt[:, 0:1], 1.0)
        pooled = jnp.concatenate([gmax, gmean], axis=1)
        out_ref[...] = _dotT(pooled, fcw[...]) + fcb[...]


_fin = pl.pallas_call(
    _fin_body,
    grid=(NBLK,),
    in_specs=[pl.BlockSpec((NB, H), lambda i: (i, 0))] * 9
    + [pl.BlockSpec((4 * H, H), lambda i: (0, 0)),
       pl.BlockSpec((1, H), lambda i: (0, 0)),
       pl.BlockSpec((NB, 1), lambda i: (i, 0)),
       pl.BlockSpec((2, 6 * H), lambda i: (0, 0)),
       pl.BlockSpec((1, 2), lambda i: (0, 0))],
    out_specs=pl.BlockSpec((G, 2), lambda i: (0, 0)),
    out_shape=_f32((G, 2)),
    scratch_shapes=[pltpu.VMEM((G, 128), jnp.float32)] * 3,
)


def kernel(x, edge_index, batch, edge_attr, conv1_w, conv1_b, conv2_w, conv2_b,
           conv3_w, conv3_b, fc_w, fc_b):
    row = edge_index[0].astype(jnp.int32)
    col = edge_index[1].astype(jnp.int32)
    ea = edge_attr.astype(jnp.float32)
    bat = batch.astype(jnp.int32)
    znN = jnp.zeros((N,), jnp.float32)
    zn8 = jnp.zeros((N, H), jnp.float32)

    d_flat, ea8 = _deg_ea8(col, ea, znN)
    d_parts = d_flat.reshape(NC, N)
    y0, z1, z2, z3, di8, dg8 = _proj(
        x, d_parts[0].reshape(N, 1), d_parts[1].reshape(N, 1),
        conv1_w.reshape(4 * H, D))

    u1_1 = _pass_first(z1, row, col, ea8, zn8)
    u1_2 = _pass_first(z2, row, col, ea8, zn8)
    u1_3 = _pass_first(z3, row, col, ea8, zn8)
    u2_2 = _pass_mid(u1_2, dg8, row, col, ea8, zn8)
    u2_3 = _pass_mid(u1_3, dg8, row, col, ea8, zn8)
    u3_3 = _pass_mid(u2_3, dg8, row, col, ea8, zn8)
    h1, g1 = _l1(y0, u1_1[0], u1_1[1], u2_2[0], u2_2[1], u3_3[0], u3_3[1],
                 di8, conv1_b.reshape(1, H))

    v1 = _pass_first(g1, row, col, ea8, zn8)
    v2 = _pass_mid(v1, dg8, row, col, ea8, zn8)
    v3 = _pass_mid(v2, dg8, row, col, ea8, zn8)
    h2, g2 = _lh(h1, v1[0], v1[1], v2[0], v2[1], v3[0], v3[1],
                 di8, conv2_w.reshape(4 * H, H), conv2_b.reshape(1, H))

    t1 = _pass_first(g2, row, col, ea8, zn8)
    t2 = _pass_mid(t1, dg8, row, col, ea8, zn8)
    t3 = _pass_mid(t2, dg8, row, col, ea8, zn8)

    out = _fin(h1, h2, t1[0], t1[1], t2[0], t2[1], t3[0], t3[1],
               di8, conv3_w.reshape(4 * H, H), conv3_b.reshape(1, H),
               bat.reshape(N, 1), fc_w, fc_b.reshape(1, 2))
    return out


# speedup vs baseline: 30.5351x; 1.0103x over previous
import functools

import jax
import jax.numpy as jnp
from jax import lax
from jax.experimental import pallas as pl
from jax.experimental.pallas import tpu as pltpu
from jax.experimental.pallas import tpu_sc as plsc

N = 10000
E = 320000
D = 128
H = 8
G = 64

NC = 2
NS = 16
NW = NC * NS
EW = E // NW
CH = 2000
NCHUNK = EW // CH
RPA = 624
TOFF = NS * RPA
TAIL = N - TOFF

NB = 1000
NBLK = N // NB

_mesh = plsc.VectorSubcoreMesh(core_axis_name="c", subcore_axis_name="s")
_sc_params = pltpu.CompilerParams(needs_layout_passes=False, use_tc_tiling_on_sc=False)


def _f32(shape):
    return jax.ShapeDtypeStruct(shape, jnp.float32)


@functools.partial(
    pl.kernel,
    out_type=[_f32((NC * N,)), _f32((E * H,))],
    mesh=_mesh,
    scratch_types=[
        pltpu.VMEM((CH,), jnp.int32),
        pltpu.VMEM((CH,), jnp.float32),
        pltpu.VMEM((CH * H,), jnp.float32),
        pltpu.VMEM((RPA,), jnp.float32),
        pltpu.VMEM_SHARED((N,), jnp.float32),
        pltpu.SemaphoreType.DMA,
    ],
    compiler_params=_sc_params,
)
def _deg_ea8(col_h, ea_h, zn_h, d_h, ea8_h, colbuf, eabuf, ea8buf, dbuf, deg_sp, sem):
    cid = lax.axis_index("c")
    sid = lax.axis_index("s")
    wid = cid * NS + sid

    @pl.when(sid == 0)
    def _():
        pltpu.sync_copy(zn_h, deg_sp)

    plsc.subcore_barrier()
    lanes = lax.iota(jnp.int32, 16)
    hi = lanes // H

    def chunk(k, carry):
        eoff = wid * EW + k * CH
        pltpu.sync_copy(col_h.at[pl.ds(eoff, CH)], colbuf)
        pltpu.sync_copy(ea_h.at[pl.ds(eoff, CH)], eabuf)

        @plsc.parallel_loop(0, CH // 2, unroll=8)
        def _(i):
            e2 = plsc.load_gather(eabuf, [hi + 2 * i])
            plsc.store_scatter(ea8buf, [lanes + 16 * i], e2)
        pltpu.sync_copy(ea8buf, ea8_h.at[pl.ds(eoff * H, CH * H)])
        pltpu.sync_copy(eabuf, deg_sp.at[colbuf], add=True)
        return carry

    lax.fori_loop(0, NCHUNK, chunk, 0)
    plsc.subcore_barrier()
    pltpu.sync_copy(deg_sp.at[pl.ds(sid * RPA, RPA)], dbuf)
    pltpu.sync_copy(dbuf, d_h.at[pl.ds(cid * N + sid * RPA, RPA)])

    @pl.when(sid == NS - 1)
    def _():
        pltpu.sync_copy(deg_sp.at[pl.ds(TOFF, TAIL)], dbuf.at[pl.ds(0, TAIL)])
        pltpu.sync_copy(dbuf.at[pl.ds(0, TAIL)],
                        d_h.at[pl.ds(cid * N + TOFF, TAIL)])


def _make_pass(two_inputs, nchains=1):
    scratch = (
        [pltpu.VMEM((CH,), jnp.int32)] * 3
        + [pltpu.VMEM((CH,), jnp.int32)] * 3
        + [pltpu.VMEM((CH * H,), jnp.float32)] * 2
        + [pltpu.VMEM((CH, H), jnp.float32)] * 2
        + [pltpu.VMEM((RPA, H), jnp.float32)] * 2
        + ([pltpu.VMEM((RPA, H), jnp.float32)] if two_inputs else [])
        + [pltpu.VMEM_SHARED((N, H), jnp.float32)] * nchains
        + [pltpu.VMEM_SHARED((N, H), jnp.float32)] * nchains
        + [pltpu.SemaphoreType.DMA] * 7
    )

    def body(*refs):
        p_hs = list(refs[:nchains])
        refs = refs[nchains:]
        if two_inputs:
            (dg8_h, row_h, col_h, ea8_h, zn8_h), refs = refs[:5], refs[5:]
        else:
            (row_h, col_h, ea8_h, zn8_h), refs = refs[:4], refs[4:]
        q_hs = list(refs[:nchains])
        rest = refs[nchains:]
        rowbufs = rest[0:3]
        colbufs = rest[3:6]
        eabufs = rest[6:8]
        gbufs = rest[8:10]
        pb0, pb1 = rest[10:12]
        nb = 13 if two_inputs else 12
        dbuf = rest[12] if two_inputs else None
        in_sps = rest[nb:nb + nchains]
        out_sps = rest[nb + nchains:nb + 2 * nchains]
        sems = rest[nb + 2 * nchains:]
        ssems = sems[0:3]
        gsems = sems[3:5]
        vsems = sems[5:7]
        cid = lax.axis_index("c")
        sid = lax.axis_index("s")
        wid = cid * NS + sid
        lanes = lax.iota(jnp.int32, 16)
        ci_base = lanes // H
        jmod = lanes % H

        @pl.when(sid == 0)
        def _():
            for c in range(nchains):
                pltpu.sync_copy(zn8_h, out_sps[c])
                if not two_inputs:
                    pltpu.sync_copy(p_hs[c], in_sps[c])

        if two_inputs:
            def prologue(c, base, rows):
                pltpu.sync_copy(p_hs[c].at[0, pl.ds(base, rows), :],
                                pb0.at[pl.ds(0, rows), :])
                pltpu.sync_copy(p_hs[c].at[1, pl.ds(base, rows), :],
                                pb1.at[pl.ds(0, rows), :])
                pltpu.sync_copy(dg8_h.at[pl.ds(base, rows), :],
                                dbuf.at[pl.ds(0, rows), :])

                @plsc.parallel_loop(0, rows // 2, unroll=8)
                def _(i):
                    ci = ci_base + 2 * i
                    v = (plsc.load_gather(pb0, [ci, jmod])
                         + plsc.load_gather(pb1, [ci, jmod]))
                    v = v * plsc.load_gather(dbuf, [ci, jmod])
                    plsc.store_scatter(pb0, [ci, jmod], v)

                pltpu.sync_copy(pb0.at[pl.ds(0, rows), :],
                                in_sps[c].at[pl.ds(base, rows), :])

            for c in range(nchains):
                prologue(c, sid * RPA, RPA)

                @pl.when(sid == NS - 1)
                def _():
                    prologue(c, TOFF, TAIL)

        plsc.subcore_barrier()

        sdesc, gdesc, vdesc = {}, {}, {}
        S = NCHUNK * nchains

        def stage(ck):
            sl = ck % 3
            eoff = wid * EW + ck * CH
            sdesc[ck] = [
                pltpu.async_copy(row_h.at[pl.ds(eoff, CH)], rowbufs[sl], ssems[sl]),
                pltpu.async_copy(col_h.at[pl.ds(eoff, CH)], colbufs[sl], ssems[sl]),
                pltpu.async_copy(ea8_h.at[pl.ds(eoff * H, CH * H)], eabufs[ck % 2],
                                 ssems[sl]),
            ]

        def gather(st):
            ck, c = st // nchains, st % nchains
            if ck in sdesc:
                for d in sdesc.pop(ck):
                    d.wait()
            gl = st % 2
            gdesc[st] = pltpu.async_copy(in_sps[c].at[rowbufs[ck % 3]],
                                         gbufs[gl], gsems[gl])

        stage(0)
        if NCHUNK > 1:
            stage(1)
        gather(0)
        for st in range(S):
            ck, c = st // nchains, st % nchains
            if st + 1 < S:
                if st >= 1:
                    vdesc.pop(st - 1).wait()
                gather(st + 1)
            gbuf = gbufs[st % 2]
            eabuf = eabufs[ck % 2]
            gdesc.pop(st).wait()

            @plsc.parallel_loop(0, CH // 2, unroll=8)
            def _(i):
                ci = ci_base + 2 * i
                g = plsc.load_gather(gbuf, [ci, jmod])
                e = plsc.load_gather(eabuf, [lanes + 16 * i])
                plsc.store_scatter(gbuf, [ci, jmod], g * e)

            vdesc[st] = pltpu.async_copy(gbuf, out_sps[c].at[colbufs[ck % 3]],
                                         vsems[st % 2], add=True)
            if c == nchains - 1 and ck + 2 < NCHUNK:
                stage(ck + 2)
        vdesc.pop(S - 2).wait()
        vdesc.pop(S - 1).wait()
        plsc.subcore_barrier()
        for c in range(nchains):
            pltpu.sync_copy(out_sps[c].at[pl.ds(sid * RPA, RPA), :], pb0)
            pltpu.sync_copy(pb0, q_hs[c].at[cid, pl.ds(sid * RPA, RPA), :])

            @pl.when(sid == NS - 1)
            def _():
                pltpu.sync_copy(out_sps[c].at[pl.ds(TOFF, TAIL), :],
                                pb1.at[pl.ds(0, TAIL), :])
                pltpu.sync_copy(pb1.at[pl.ds(0, TAIL), :],
                                q_hs[c].at[cid, pl.ds(TOFF, TAIL), :])

    out_type = [_f32((NC, N, H))] * nchains
    if nchains == 1:
        out_type = out_type[0]
    return pl.kernel(body, out_type=out_type, mesh=_mesh,
                     scratch_types=scratch, compiler_params=_sc_params)


_pass_first = _make_pass(False)
_pass_mid = _make_pass(True)
_pass_first3 = _make_pass(False, nchains=3)
_pass_mid2 = _make_pass(True, nchains=2)


def _proj_body(x_ref, d0_ref, d1_ref, w_ref, y0_ref, z1_ref, z2_ref, z3_ref,
               di8_ref, dg8_ref):
    deg = d0_ref[...] + d1_ref[...]
    dinv = jnp.where(deg > 0, lax.rsqrt(jnp.maximum(deg, 1e-12)), 0.0)
    dg = dinv * dinv
    yz = lax.dot_general(x_ref[...], w_ref[...], (((1,), (1,)), ((), ())),
                         preferred_element_type=jnp.float32)
    y0_ref[...] = yz[:, 0:H]
    z1_ref[...] = dinv * yz[:, H:2 * H]
    z2_ref[...] = dinv * yz[:, 2 * H:3 * H]
    z3_ref[...] = dinv * yz[:, 3 * H:4 * H]
    di8_ref[...] = jnp.broadcast_to(dinv, (NB, H))
    dg8_ref[...] = jnp.broadcast_to(dg, (NB, H))


_proj = pl.pallas_call(
    _proj_body,
    grid=(NBLK,),
    in_specs=[
        pl.BlockSpec((NB, D), lambda i: (i, 0)),
        pl.BlockSpec((NB, 1), lambda i: (i, 0)),
        pl.BlockSpec((NB, 1), lambda i: (i, 0)),
        pl.BlockSpec((4 * H, D), lambda i: (0, 0)),
    ],
    out_specs=[pl.BlockSpec((NB, H), lambda i: (i, 0))] * 6,
    out_shape=[_f32((N, H))] * 6,
)


def _l1_body(y0, u1a, u1b, u2a, u2b, u3a, u3b, di8, b, h_ref, g_ref):
    s = (u1a[...] + u1b[...]) + (u2a[...] + u2b[...]) + (u3a[...] + u3b[...])
    h = jnp.maximum(y0[...] + di8[...] * s + b[...], 0.0)
    h_ref[...] = h
    g_ref[...] = di8[...] * h


_l1 = pl.pallas_call(
    _l1_body,
    grid=(NBLK,),
    in_specs=[pl.BlockSpec((NB, H), lambda i: (i, 0))] * 8
    + [pl.BlockSpec((1, H), lambda i: (0, 0))],
    out_specs=[pl.BlockSpec((NB, H), lambda i: (i, 0))] * 2,
    out_shape=[_f32((N, H))] * 2,
)


def _dotT(a, w):
    return lax.dot_general(a, w, (((1,), (1,)), ((), ())),
                           preferred_element_type=jnp.float32)


def _lh_body(hp, v1a, v1b, v2a, v2b, v3a, v3b, di8, w_ref, b, h_ref, g_ref):
    w = w_ref[...]
    t = (_dotT(v1a[...] + v1b[...], w[H:2 * H])
         + _dotT(v2a[...] + v2b[...], w[2 * H:3 * H])
         + _dotT(v3a[...] + v3b[...], w[3 * H:4 * H]))
    h = jnp.maximum(_dotT(hp[...], w[0:H]) + di8[...] * t + b[...], 0.0)
    h_ref[...] = h
    g_ref[...] = di8[...] * h


_lh = pl.pallas_call(
    _lh_body,
    grid=(NBLK,),
    in_specs=[pl.BlockSpec((NB, H), lambda i: (i, 0))] * 8
    + [pl.BlockSpec((4 * H, H), lambda i: (0, 0)),
       pl.BlockSpec((1, H), lambda i: (0, 0))],
    out_specs=[pl.BlockSpec((NB, H), lambda i: (i, 0))] * 2,
    out_shape=[_f32((N, H))] * 2,
)


def _fin_body(h1r, h2r, v1a, v1b, v2a, v2b, v3a, v3b, di8, w_ref, b, batr,
              fcw, fcb, out_ref, smax, ssum, scnt):
    i = pl.program_id(0)

    @pl.when(i == 0)
    def _():
        smax[...] = jnp.full((G, 128), -jnp.inf, jnp.float32)
        ssum[...] = jnp.zeros((G, 128), jnp.float32)
        scnt[...] = jnp.zeros((G, 128), jnp.float32)

    w = w_ref[...]
    t = (_dotT(v1a[...] + v1b[...], w[H:2 * H])
         + _dotT(v2a[...] + v2b[...], w[2 * H:3 * H])
         + _dotT(v3a[...] + v3b[...], w[3 * H:4 * H]))
    h3 = jnp.maximum(_dotT(h2r[...], w[0:H]) + di8[...] * t + b[...], 0.0)
    hcat = jnp.concatenate([h1r[...], h2r[...], h3], axis=1)

    bb = batr[...]
    gi = lax.broadcasted_iota(jnp.int32, (NB, G), 1)
    oh = (bb == gi).astype(jnp.float32)
    ssum[:, 0:3 * H] = ssum[:, 0:3 * H] + lax.dot_general(
        oh, hcat, (((0,), (0,)), ((), ())), preferred_element_type=jnp.float32)
    scnt[:, 0:1] = scnt[:, 0:1] + lax.dot_general(
        oh, jnp.ones((NB, 1), jnp.float32), (((0,), (0,)), ((), ())),
        preferred_element_type=jnp.float32)

    neg = jnp.float32(-jnp.inf)
    for g in range(G):
        m = bb == g
        rmax = jnp.max(jnp.where(m, hcat, neg), axis=0, keepdims=True)
        smax[g:g + 1, 0:3 * H] = jnp.maximum(smax[g:g + 1, 0:3 * H], rmax)

    @pl.when(i == NBLK - 1)
    def _():
        gmax = smax[:, 0:3 * H]
        gmax = jnp.where(gmax > jnp.float32(-3e38), gmax, 0.0)
        gmean = ssum[:, 0:3 * H] / jnp.maximum(scnt[:, 0:1], 1.0)
        pooled = jnp.concatenate([gmax, gmean], axis=1)
        out_ref[...] = _dotT(pooled, fcw[...]) + fcb[...]


_fin = pl.pallas_call(
    _fin_body,
    grid=(NBLK,),
    in_specs=[pl.BlockSpec((NB, H), lambda i: (i, 0))] * 9
    + [pl.BlockSpec((4 * H, H), lambda i: (0, 0)),
       pl.BlockSpec((1, H), lambda i: (0, 0)),
       pl.BlockSpec((NB, 1), lambda i: (i, 0)),
       pl.BlockSpec((2, 6 * H), lambda i: (0, 0)),
       pl.BlockSpec((1, 2), lambda i: (0, 0))],
    out_specs=pl.BlockSpec((G, 2), lambda i: (0, 0)),
    out_shape=_f32((G, 2)),
    scratch_shapes=[pltpu.VMEM((G, 128), jnp.float32)] * 3,
)


def kernel(x, edge_index, batch, edge_attr, conv1_w, conv1_b, conv2_w, conv2_b,
           conv3_w, conv3_b, fc_w, fc_b):
    row = edge_index[0].astype(jnp.int32)
    col = edge_index[1].astype(jnp.int32)
    ea = edge_attr.astype(jnp.float32)
    bat = batch.astype(jnp.int32)
    znN = jnp.zeros((N,), jnp.float32)
    zn8 = jnp.zeros((N, H), jnp.float32)

    d_flat, ea8 = _deg_ea8(col, ea, znN)
    d_parts = d_flat.reshape(NC, N)
    y0, z1, z2, z3, di8, dg8 = _proj(
        x, d_parts[0].reshape(N, 1), d_parts[1].reshape(N, 1),
        conv1_w.reshape(4 * H, D))

    u1_1, u1_2, u1_3 = _pass_first3(z1, z2, z3, row, col, ea8, zn8)
    u2_2, u2_3 = _pass_mid2(u1_2, u1_3, dg8, row, col, ea8, zn8)
    u3_3 = _pass_mid(u2_3, dg8, row, col, ea8, zn8)
    h1, g1 = _l1(y0, u1_1[0], u1_1[1], u2_2[0], u2_2[1], u3_3[0], u3_3[1],
                 di8, conv1_b.reshape(1, H))

    v1 = _pass_first(g1, row, col, ea8, zn8)
    v2 = _pass_mid(v1, dg8, row, col, ea8, zn8)
    v3 = _pass_mid(v2, dg8, row, col, ea8, zn8)
    h2, g2 = _lh(h1, v1[0], v1[1], v2[0], v2[1], v3[0], v3[1],
                 di8, conv2_w.reshape(4 * H, H), conv2_b.reshape(1, H))

    t1 = _pass_first(g2, row, col, ea8, zn8)
    t2 = _pass_mid(t1, dg8, row, col, ea8, zn8)
    t3 = _pass_mid(t2, dg8, row, col, ea8, zn8)

    out = _fin(h1, h2, t1[0], t1[1], t2[0], t2[1], t3[0], t3[1],
               di8, conv3_w.reshape(4 * H, H), conv3_b.reshape(1, H),
               bat.reshape(N, 1), fc_w, fc_b.reshape(1, 2))
    return out
